# baseline (speedup 1.0000x reference)
import functools

import jax
import jax.numpy as jnp
from jax import lax
from jax.experimental import pallas as pl
from jax.experimental.pallas import tpu as pltpu
from jax.experimental.pallas import tpu_sc as plsc

F32 = jnp.float32

NCORE = 2
NSUB = 16
NW = NCORE * NSUB

CHUNK = 100


def _deg_partial(dst_rs, ones_flat, zflat, npad):
    ngroup, gchunk = dst_rs.shape[1], dst_rs.shape[2]
    rpt = npad // NSUB
    mesh = plsc.VectorSubcoreMesh(core_axis_name="c", subcore_axis_name="s")

    @functools.partial(
        pl.kernel,
        out_type=jax.ShapeDtypeStruct((NCORE * npad,), F32),
        mesh=mesh,
        scratch_types=[
            pltpu.VMEM((gchunk, CHUNK), jnp.int32),
            pltpu.VMEM((CHUNK,), F32),
            pltpu.VMEM_SHARED((npad,), F32),
        ],
    )
    def k(dst_hbm, ones_hbm, z_hbm, out_hbm, idx_v, ones_v, degsh):
        cid = lax.axis_index("c")
        sid = lax.axis_index("s")
        wid = cid * NSUB + sid

        pltpu.sync_copy(ones_hbm, ones_v)
        pltpu.sync_copy(z_hbm, degsh.at[pl.ds(sid * rpt, rpt)])
        plsc.subcore_barrier()

        def group(g, carry):
            pltpu.sync_copy(dst_hbm.at[wid, g], idx_v)

            def body(j, c2):
                pltpu.sync_copy(ones_v, degsh.at[idx_v.at[j]], add=True)
                return c2

            lax.fori_loop(0, gchunk, body, 0)
            return carry

        lax.fori_loop(0, ngroup, group, 0)

        plsc.subcore_barrier()
        pltpu.sync_copy(degsh.at[pl.ds(sid * rpt, rpt)],
                        out_hbm.at[pl.ds(cid * npad + sid * rpt, rpt)])

    return k(dst_rs, ones_flat, zflat).reshape(NCORE, npad)


def _matmul_scale(x, W, dp):
    n, d_in = x.shape
    d_out = W.shape[1]
    npad = dp.shape[1]
    bn = 512
    grid = (n // bn,)

    def body(x_ref, w_ref, dp_ref, hn_ref):
        i = pl.program_id(0)
        h = jnp.dot(x_ref[...], w_ref[...], preferred_element_type=F32)
        dpb = dp_ref[:, pl.ds(i * bn, bn)]
        deg = dpb[0] + dpb[1] + 1.0
        inv = lax.rsqrt(deg)
        hn_ref[...] = h * inv[:, None]

    return pl.pallas_call(
        body,
        grid=grid,
        in_specs=[
            pl.BlockSpec((bn, d_in), lambda i: (i, 0)),
            pl.BlockSpec((d_in, d_out), lambda i: (0, 0)),
            pl.BlockSpec((NCORE, npad), lambda i: (0, 0)),
        ],
        out_specs=pl.BlockSpec((bn, d_out), lambda i: (i, 0)),
        out_shape=jax.ShapeDtypeStruct((n, d_out), F32),
    )(x, W, dp)


def _edge_pass(hn, src_rs, dst_rs, zrows, npad):
    d = hn.shape[1]
    n = npad
    ngroup, gchunk = src_rs.shape[1], src_rs.shape[2]
    rpt = n // NSUB
    mesh = plsc.VectorSubcoreMesh(core_axis_name="c", subcore_axis_name="s")

    nbuf = 3
    @functools.partial(
        pl.kernel,
        out_type=jax.ShapeDtypeStruct((NCORE, n, d), F32),
        mesh=mesh,
        scratch_types=[
            pltpu.VMEM((gchunk, CHUNK), jnp.int32),
            pltpu.VMEM((gchunk, CHUNK), jnp.int32),
            [pltpu.VMEM((CHUNK, d), F32) for _ in range(nbuf)],
            [pltpu.SemaphoreType.DMA for _ in range(nbuf)],
            [pltpu.SemaphoreType.DMA for _ in range(nbuf)],
            pltpu.VMEM_SHARED((n, d), F32),
        ],
    )
    def k(hn_hbm, src_hbm, dst_hbm, z_hbm, out_hbm,
          src_v, dst_v, rows, gsem, ssem, agg):
        cid = lax.axis_index("c")
        sid = lax.axis_index("s")
        wid = cid * NSUB + sid

        pltpu.sync_copy(z_hbm, agg.at[pl.ds(sid * rpt, rpt)])
        plsc.subcore_barrier()

        def start_g(j, b):
            pltpu.async_copy(hn_hbm.at[src_v.at[j]], rows[b], gsem[b])

        def wait_g(j, b):
            pltpu.make_async_copy(
                hn_hbm.at[src_v.at[j]], rows[b], gsem[b]).wait()

        def start_s(j, b):
            pltpu.async_copy(rows[b], agg.at[dst_v.at[j]], ssem[b],
                             add=True)

        def wait_s(j, b):
            pltpu.make_async_copy(
                rows[b], agg.at[dst_v.at[j]], ssem[b]).wait()

        def group(g, carry):
            pltpu.sync_copy(src_hbm.at[wid, g], src_v)
            pltpu.sync_copy(dst_hbm.at[wid, g], dst_v)
            start_g(0, 0)
            start_g(1, 1)
            wait_g(0, 0); start_s(0, 0); start_g(2, 2)

            def body(t, c2):
                for b3 in range(3):
                    j = 1 + 3 * t + b3
                    b = (1 + b3) % 3
                    wait_g(j, b)
                    start_s(j, b)
                    wait_s(j - 1, b3)
                    start_g(j + 2, b3)
                return c2

            lax.fori_loop(0, (gchunk - 5) // 3, body, 0)

            jt = gchunk - 4
            wait_g(jt, 1); start_s(jt, 1); wait_s(jt - 1, 0); start_g(jt + 2, 0)
            wait_g(jt + 1, 2); start_s(jt + 1, 2); wait_s(jt, 1); start_g(jt + 3, 1)
            wait_g(jt + 2, 0); start_s(jt + 2, 0); wait_s(jt + 1, 2)
            wait_g(jt + 3, 1); start_s(jt + 3, 1)
            wait_s(jt + 2, 0)
            wait_s(jt + 3, 1)
            return carry

        lax.fori_loop(0, ngroup, group, 0)

        plsc.subcore_barrier()
        pltpu.sync_copy(agg.at[pl.ds(sid * rpt, rpt)],
                        out_hbm.at[cid, pl.ds(sid * rpt, rpt)])

    return k(hn, src_rs, dst_rs, zrows)


def _final(agg, hn, dp, b2):
    n, d = hn.shape
    npad = agg.shape[1]
    bn = 512
    grid = (n // bn,)

    def body(agg_ref, hn_ref, dp_ref, b_ref, o_ref):
        i = pl.program_id(0)
        dpb = dp_ref[:, pl.ds(i * bn, bn)]
        deg = dpb[0] + dpb[1] + 1.0
        inv = lax.rsqrt(deg)
        a = agg_ref[0] + agg_ref[1] + hn_ref[...]
        o_ref[...] = a * inv[:, None] + b_ref[...]

    return pl.pallas_call(
        body,
        grid=grid,
        in_specs=[
            pl.BlockSpec((NCORE, bn, d), lambda i: (0, i, 0)),
            pl.BlockSpec((bn, d), lambda i: (i, 0)),
            pl.BlockSpec((NCORE, npad), lambda i: (0, 0)),
            pl.BlockSpec((1, d), lambda i: (0, 0)),
        ],
        out_specs=pl.BlockSpec((bn, d), lambda i: (i, 0)),
        out_shape=jax.ShapeDtypeStruct((n, d), F32),
    )(agg, hn, dp, b2)


def kernel(x, edge_index, W, b):
    n, d_in = x.shape
    d_out = W.shape[1]
    e = edge_index.shape[1]
    ept = e // NW
    nchunk = ept // CHUNK

    npad = ((n + 128 * NSUB - 1) // (128 * NSUB)) * (128 * NSUB)

    ngroup = 5
    ei = edge_index.astype(jnp.int32)
    src_rs = ei[0].reshape(NW, ngroup, nchunk // ngroup, CHUNK)
    dst_rs = ei[1].reshape(NW, ngroup, nchunk // ngroup, CHUNK)
    zrows = jnp.zeros((npad // NSUB, d_out), F32)
    ones_flat = jnp.ones((CHUNK,), F32)
    zflat = jnp.zeros((npad // NSUB,), F32)
    b2 = b.reshape(1, d_out)

    xp = jnp.pad(x, ((0, npad - n), (0, 0)))
    dp = _deg_partial(dst_rs, ones_flat, zflat, npad)
    hn = _matmul_scale(xp, W, dp)
    agg = _edge_pass(hn, src_rs, dst_rs, zrows, npad)
    out = _final(agg, hn, dp, b2)
    return out[:n]


# speedup vs baseline: 1.0139x; 1.0139x over previous
import functools

import jax
import jax.numpy as jnp
from jax import lax
from jax.experimental import pallas as pl
from jax.experimental.pallas import tpu as pltpu
from jax.experimental.pallas import tpu_sc as plsc

F32 = jnp.float32

NCORE = 2
NSUB = 16
NW = NCORE * NSUB

CHUNK = 100


def _deg_partial(dst_rs, ones_flat, zflat, npad):
    ngroup, gchunk = dst_rs.shape[1], dst_rs.shape[2]
    rpt = npad // NSUB
    mesh = plsc.VectorSubcoreMesh(core_axis_name="c", subcore_axis_name="s")

    @functools.partial(
        pl.kernel,
        out_type=jax.ShapeDtypeStruct((NCORE * npad,), F32),
        mesh=mesh,
        scratch_types=[
            pltpu.VMEM((ngroup, gchunk, CHUNK), jnp.int32),
            pltpu.VMEM((CHUNK,), F32),
            pltpu.VMEM_SHARED((npad,), F32),
        ],
    )
    def k(dst_hbm, ones_hbm, z_hbm, out_hbm, idx_v, ones_v, degsh):
        cid = lax.axis_index("c")
        sid = lax.axis_index("s")
        wid = cid * NSUB + sid

        pltpu.sync_copy(dst_hbm.at[wid], idx_v)
        pltpu.sync_copy(ones_hbm, ones_v)
        pltpu.sync_copy(z_hbm, degsh.at[pl.ds(sid * rpt, rpt)])
        plsc.subcore_barrier()

        def group(g, carry):
            def body(j, c2):
                pltpu.sync_copy(ones_v, degsh.at[idx_v.at[g, j]], add=True)
                return c2
            lax.fori_loop(0, gchunk, body, 0)
            return carry

        lax.fori_loop(0, ngroup, group, 0)

        plsc.subcore_barrier()
        pltpu.sync_copy(degsh.at[pl.ds(sid * rpt, rpt)],
                        out_hbm.at[pl.ds(cid * npad + sid * rpt, rpt)])

    return k(dst_rs, ones_flat, zflat).reshape(NCORE, npad)


def _matmul_scale(x, W, dp):
    n, d_in = x.shape
    d_out = W.shape[1]
    npad = dp.shape[1]
    bn = 512
    grid = (n // bn,)

    def body(x_ref, w_ref, dp_ref, hn_ref):
        i = pl.program_id(0)
        h = jnp.dot(x_ref[...], w_ref[...], preferred_element_type=F32)
        dpb = dp_ref[:, pl.ds(i * bn, bn)]
        deg = dpb[0] + dpb[1] + 1.0
        inv = lax.rsqrt(deg)
        hn_ref[...] = h * inv[:, None]

    return pl.pallas_call(
        body,
        grid=grid,
        in_specs=[
            pl.BlockSpec((bn, d_in), lambda i: (i, 0)),
            pl.BlockSpec((d_in, d_out), lambda i: (0, 0)),
            pl.BlockSpec((NCORE, npad), lambda i: (0, 0)),
        ],
        out_specs=pl.BlockSpec((bn, d_out), lambda i: (i, 0)),
        out_shape=jax.ShapeDtypeStruct((n, d_out), F32),
    )(x, W, dp)


def _edge_pass(hn, src_rs, dst_rs, zrows, npad):
    d = hn.shape[1]
    n = npad
    ngroup, gchunk = src_rs.shape[1], src_rs.shape[2]
    rpt = n // NSUB
    mesh = plsc.VectorSubcoreMesh(core_axis_name="c", subcore_axis_name="s")

    nbuf = 3
    @functools.partial(
        pl.kernel,
        out_type=jax.ShapeDtypeStruct((NCORE, n, d), F32),
        mesh=mesh,
        scratch_types=[
            pltpu.VMEM((gchunk, CHUNK), jnp.int32),
            pltpu.VMEM((gchunk, CHUNK), jnp.int32),
            [pltpu.VMEM((CHUNK, d), F32) for _ in range(nbuf)],
            [pltpu.SemaphoreType.DMA for _ in range(nbuf)],
            [pltpu.SemaphoreType.DMA for _ in range(nbuf)],
            pltpu.VMEM_SHARED((n, d), F32),
        ],
    )
    def k(hn_hbm, src_hbm, dst_hbm, z_hbm, out_hbm,
          src_v, dst_v, rows, gsem, ssem, agg):
        cid = lax.axis_index("c")
        sid = lax.axis_index("s")
        wid = cid * NSUB + sid

        pltpu.sync_copy(z_hbm, agg.at[pl.ds(sid * rpt, rpt)])
        plsc.subcore_barrier()

        def start_g(j, b):
            pltpu.async_copy(hn_hbm.at[src_v.at[j]], rows[b], gsem[b])

        def wait_g(j, b):
            pltpu.make_async_copy(
                hn_hbm.at[src_v.at[j]], rows[b], gsem[b]).wait()

        def start_s(j, b):
            pltpu.async_copy(rows[b], agg.at[dst_v.at[j]], ssem[b],
                             add=True)

        def wait_s(j, b):
            pltpu.make_async_copy(
                rows[b], agg.at[dst_v.at[j]], ssem[b]).wait()

        def group(g, carry):
            pltpu.sync_copy(src_hbm.at[wid, g], src_v)
            pltpu.sync_copy(dst_hbm.at[wid, g], dst_v)
            start_g(0, 0)
            start_g(1, 1)
            wait_g(0, 0); start_s(0, 0); start_g(2, 2)

            def body(t, c2):
                for b3 in range(3):
                    j = 1 + 3 * t + b3
                    b = (1 + b3) % 3
                    wait_g(j, b)
                    start_s(j, b)
                    wait_s(j - 1, b3)
                    start_g(j + 2, b3)
                return c2

            lax.fori_loop(0, (gchunk - 5) // 3, body, 0)

            jt = gchunk - 4
            wait_g(jt, 1); start_s(jt, 1); wait_s(jt - 1, 0); start_g(jt + 2, 0)
            wait_g(jt + 1, 2); start_s(jt + 1, 2); wait_s(jt, 1); start_g(jt + 3, 1)
            wait_g(jt + 2, 0); start_s(jt + 2, 0); wait_s(jt + 1, 2)
            wait_g(jt + 3, 1); start_s(jt + 3, 1)
            wait_s(jt + 2, 0)
            wait_s(jt + 3, 1)
            return carry

        lax.fori_loop(0, ngroup, group, 0)

        plsc.subcore_barrier()
        pltpu.sync_copy(agg.at[pl.ds(sid * rpt, rpt)],
                        out_hbm.at[cid, pl.ds(sid * rpt, rpt)])

    return k(hn, src_rs, dst_rs, zrows)


def _final(agg, hn, dp, b2):
    n, d = hn.shape
    npad = agg.shape[1]
    bn = 512
    grid = (n // bn,)

    def body(agg_ref, hn_ref, dp_ref, b_ref, o_ref):
        i = pl.program_id(0)
        dpb = dp_ref[:, pl.ds(i * bn, bn)]
        deg = dpb[0] + dpb[1] + 1.0
        inv = lax.rsqrt(deg)
        a = agg_ref[0] + agg_ref[1] + hn_ref[...]
        o_ref[...] = a * inv[:, None] + b_ref[...]

    return pl.pallas_call(
        body,
        grid=grid,
        in_specs=[
            pl.BlockSpec((NCORE, bn, d), lambda i: (0, i, 0)),
            pl.BlockSpec((bn, d), lambda i: (i, 0)),
            pl.BlockSpec((NCORE, npad), lambda i: (0, 0)),
            pl.BlockSpec((1, d), lambda i: (0, 0)),
        ],
        out_specs=pl.BlockSpec((bn, d), lambda i: (i, 0)),
        out_shape=jax.ShapeDtypeStruct((n, d), F32),
    )(agg, hn, dp, b2)


def kernel(x, edge_index, W, b):
    n, d_in = x.shape
    d_out = W.shape[1]
    e = edge_index.shape[1]
    ept = e // NW
    nchunk = ept // CHUNK

    npad = ((n + 128 * NSUB - 1) // (128 * NSUB)) * (128 * NSUB)

    ngroup = 5
    ei = edge_index.astype(jnp.int32)
    src_rs = ei[0].reshape(NW, ngroup, nchunk // ngroup, CHUNK)
    dst_rs = ei[1].reshape(NW, ngroup, nchunk // ngroup, CHUNK)
    zrows = jnp.zeros((npad // NSUB, d_out), F32)
    ones_flat = jnp.ones((CHUNK,), F32)
    zflat = jnp.zeros((npad // NSUB,), F32)
    b2 = b.reshape(1, d_out)

    xp = jnp.pad(x, ((0, npad - n), (0, 0)))
    dp = _deg_partial(dst_rs, ones_flat, zflat, npad)
    hn = _matmul_scale(xp, W, dp)
    agg = _edge_pass(hn, src_rs, dst_rs, zrows, npad)
    out = _final(agg, hn, dp, b2)
    return out[:n]
